# baseline (speedup 1.0000x reference)
import functools

import jax
import jax.numpy as jnp
from jax import lax
from jax.experimental import pallas as pl
from jax.experimental.pallas import tpu as pltpu
from jax.experimental.pallas import tpu_sc as plsc

B = 1024
N = 20000
N_DN = 1300
N_MBON = 96
L = 16
NC = 2
NS = 16
NW = NC * NS
BLK = 8
BLKS_PER_W = B // BLK // NW

SPLIT_W = 4992
N_FULL_SPLITS = 4
TAIL_OFF = SPLIT_W * N_FULL_SPLITS
TAIL_W = N - TAIL_OFF
OFFS = (0, 4992, 9984, 14976, 19968)
WIDTHS = (4992, 4992, 4992, 4992, TAIL_W)
NSEG = 10
CAP = 1552


def _seg_scalar(vec16, i):
    return jnp.sum(jnp.where(lax.iota(jnp.int32, L) == i, vec16, 0))


def _sc_body(v_hbm, cidx_hbm, cpos_hbm, scal_hbm, dn_out_hbm, mbon_out_hbm,
             cidx_v, cpos_v, scal_v, vb, vt, dnb, mbb):
    wid = lax.axis_index("s") * NC + lax.axis_index("c")

    pltpu.sync_copy(cidx_hbm, cidx_v)
    pltpu.sync_copy(cpos_hbm, cpos_v)
    pltpu.sync_copy(scal_hbm, scal_v)

    sv0 = scal_v[pl.ds(0, L)]
    sv1 = scal_v[pl.ds(L, L)]
    astart = [_seg_scalar(sv0, i) for i in range(NSEG + 1)]
    aend = [_seg_scalar(sv1, i) for i in range(NSEG)]

    def block_body(k, carry):
        rb = wid * BLKS_PER_W + k
        r0 = rb * BLK

        for s in range(len(OFFS)):
            buf = vt if s == len(OFFS) - 1 else vb
            pltpu.sync_copy(
                v_hbm.at[pl.ds(r0, BLK), pl.ds(OFFS[s], WIDTHS[s])], buf)
            for half in range(2):
                seg = 2 * s + half
                outb = dnb if half == 0 else mbb
                nch = (astart[seg + 1] - astart[seg]) // L

                def chunk_body(j, c, seg=seg, buf=buf, outb=outb):
                    p = astart[seg] + j * L
                    cvec = cidx_v[pl.ds(p, L)]
                    pvec = cpos_v[pl.ds(p, L)]
                    mask = (p + lax.iota(jnp.int32, L)) < aend[seg]
                    for r in range(BLK):
                        rvec = jnp.full((L,), r, jnp.int32)
                        g = plsc.load_gather(buf, [rvec, cvec])
                        plsc.store_scatter(outb, [rvec, pvec], g, mask=mask)
                    return c

                lax.fori_loop(0, nch, chunk_body, 0)

        pltpu.sync_copy(dnb, dn_out_hbm.at[pl.ds(r0, BLK)])
        pltpu.sync_copy(mbb, mbon_out_hbm.at[pl.ds(r0, BLK)])
        return carry

    lax.fori_loop(0, BLKS_PER_W, block_body, 0)


@jax.jit
def kernel(v, dn_id, mbon_id):
    dn_id = dn_id.astype(jnp.int32)
    mbon_id = mbon_id.astype(jnp.int32)

    all_idx = jnp.concatenate([dn_id, mbon_id])
    ar = jnp.arange(N_DN + N_MBON, dtype=jnp.int32)
    is_mbon = ar >= N_DN
    outpos = jnp.where(is_mbon, ar - N_DN, ar)
    region = jnp.minimum(all_idx // SPLIT_W, N_FULL_SPLITS)
    seg = region * 2 + is_mbon.astype(jnp.int32)

    order = jnp.argsort(seg, stable=True)
    sseg = seg[order]
    slocal = (all_idx - jnp.asarray(OFFS, jnp.int32)[region])[order]
    spos = outpos[order]

    cnt = jnp.sum(sseg[None, :] == jnp.arange(NSEG)[:, None], axis=1)
    aligned = ((cnt + L - 1) // L) * L
    astart = jnp.concatenate([jnp.zeros((1,), jnp.int32),
                              jnp.cumsum(aligned).astype(jnp.int32)])
    sstart = jnp.concatenate([jnp.zeros((1,), jnp.int32),
                              jnp.cumsum(cnt).astype(jnp.int32)])[:-1]
    rank = ar - sstart[sseg]
    target = astart[sseg] + rank

    cidx = jnp.zeros((CAP,), jnp.int32).at[target].set(slocal)
    cpos = jnp.zeros((CAP,), jnp.int32).at[target].set(spos)
    aend = astart[:NSEG] + cnt.astype(jnp.int32)
    scal = jnp.zeros((2 * L,), jnp.int32)
    scal = scal.at[0:NSEG + 1].set(astart)
    scal = scal.at[L:L + NSEG].set(aend)

    mesh = plsc.VectorSubcoreMesh(core_axis_name="c", subcore_axis_name="s")
    run = pl.kernel(
        _sc_body,
        mesh=mesh,
        compiler_params=pltpu.CompilerParams(needs_layout_passes=False,
                                             use_tc_tiling_on_sc=True),
        out_type=(jax.ShapeDtypeStruct((B, N_DN), jnp.float32),
                  jax.ShapeDtypeStruct((B, N_MBON), jnp.float32)),
        scratch_types=[
            pltpu.VMEM((CAP,), jnp.int32),
            pltpu.VMEM((CAP,), jnp.int32),
            pltpu.VMEM((2 * L,), jnp.int32),
            pltpu.VMEM((BLK, SPLIT_W), jnp.float32),
            pltpu.VMEM((BLK, TAIL_W), jnp.float32),
            pltpu.VMEM((BLK, N_DN), jnp.float32),
            pltpu.VMEM((BLK, N_MBON), jnp.float32),
        ],
    )
    return run(v, cidx, cpos, scal)


# speedup vs baseline: 7.7488x; 7.7488x over previous
import functools

import jax
import jax.numpy as jnp
from jax import lax
from jax.experimental import pallas as pl
from jax.experimental.pallas import tpu as pltpu
from jax.experimental.pallas import tpu_sc as plsc

B = 1024
N = 20000
N_DN = 1300
N_MBON = 96
NC = 2
NS = 16
NW = NC * NS
BLK = 8
DN_BLKS = (N_DN + BLK - 1) // BLK
DN_TAIL = N_DN - (DN_BLKS - 1) * BLK
MB_BLKS = N_MBON // BLK
TOT_BLKS = DN_BLKS + MB_BLKS
IDX_PAD = TOT_BLKS * BLK
MAX_BLKS_PER_W = (TOT_BLKS + NW - 1) // NW
NBUF = 3


def _sc_body(vt_hbm, cidx_hbm, dnt_hbm, mbt_hbm,
             cidx_v, g0, g1, g2, sg0, sg1, sg2, so0, so1, so2):
    wid = lax.axis_index("s") * NC + lax.axis_index("c")
    gb = (g0, g1, g2)
    sg = (sg0, sg1, sg2)
    so = (so0, so1, so2)

    pltpu.sync_copy(cidx_hbm, cidx_v)

    def blk_of(k):
        return wid + NW * k

    def issue_gather(k):
        blk = blk_of(k)

        @pl.when(blk < TOT_BLKS)
        def _():
            pltpu.async_copy(vt_hbm.at[cidx_v.at[pl.ds(blk * BLK, BLK)]],
                             gb[k % NBUF], sg[k % NBUF])

    def wait_gather(k):
        blk = blk_of(k)

        @pl.when(blk < TOT_BLKS)
        def _():
            pltpu.make_async_copy(
                vt_hbm.at[cidx_v.at[pl.ds(blk * BLK, BLK)]],
                gb[k % NBUF], sg[k % NBUF]).wait()

    def out_copies(k, blk):
        full_dn = blk < DN_BLKS - 1
        part_dn = blk == DN_BLKS - 1
        is_mb = (blk >= DN_BLKS) & (blk < TOT_BLKS)
        return full_dn, part_dn, is_mb

    def issue_out(k):
        blk = blk_of(k)
        full_dn, part_dn, is_mb = out_copies(k, blk)

        @pl.when(full_dn)
        def _():
            pltpu.async_copy(gb[k % NBUF], dnt_hbm.at[pl.ds(blk * BLK, BLK)],
                             so[k % NBUF])

        @pl.when(part_dn)
        def _():
            pltpu.async_copy(gb[k % NBUF].at[pl.ds(0, DN_TAIL)],
                             dnt_hbm.at[pl.ds((DN_BLKS - 1) * BLK, DN_TAIL)],
                             so[k % NBUF])

        @pl.when(is_mb)
        def _():
            pltpu.async_copy(gb[k % NBUF],
                             mbt_hbm.at[pl.ds((blk - DN_BLKS) * BLK, BLK)],
                             so[k % NBUF])

    def wait_out(k):
        blk = blk_of(k)
        full_dn, part_dn, is_mb = out_copies(k, blk)

        @pl.when(full_dn)
        def _():
            pltpu.make_async_copy(gb[k % NBUF],
                                  dnt_hbm.at[pl.ds(blk * BLK, BLK)],
                                  so[k % NBUF]).wait()

        @pl.when(part_dn)
        def _():
            pltpu.make_async_copy(
                gb[k % NBUF].at[pl.ds(0, DN_TAIL)],
                dnt_hbm.at[pl.ds((DN_BLKS - 1) * BLK, DN_TAIL)],
                so[k % NBUF]).wait()

        @pl.when(is_mb)
        def _():
            pltpu.make_async_copy(gb[k % NBUF],
                                  mbt_hbm.at[pl.ds((blk - DN_BLKS) * BLK, BLK)],
                                  so[k % NBUF]).wait()

    for k in range(min(NBUF, MAX_BLKS_PER_W)):
        issue_gather(k)
    for k in range(MAX_BLKS_PER_W):
        if k >= NBUF:
            wait_out(k - NBUF)
            issue_gather(k)
        wait_gather(k)
        issue_out(k)
    for k in range(max(0, MAX_BLKS_PER_W - NBUF), MAX_BLKS_PER_W):
        wait_out(k)


@jax.jit
def kernel(v, dn_id, mbon_id):
    vt = jnp.swapaxes(v, 0, 1)
    cidx = jnp.concatenate(
        [dn_id.astype(jnp.int32),
         jnp.zeros((DN_BLKS * BLK - N_DN,), jnp.int32),
         mbon_id.astype(jnp.int32)])

    mesh = plsc.VectorSubcoreMesh(core_axis_name="c", subcore_axis_name="s")
    run = pl.kernel(
        _sc_body,
        mesh=mesh,
        compiler_params=pltpu.CompilerParams(needs_layout_passes=False,
                                             use_tc_tiling_on_sc=True),
        out_type=(jax.ShapeDtypeStruct((N_DN, B), jnp.float32),
                  jax.ShapeDtypeStruct((N_MBON, B), jnp.float32)),
        scratch_types=(
            [pltpu.VMEM((IDX_PAD,), jnp.int32)]
            + [pltpu.VMEM((BLK, B), jnp.float32) for _ in range(NBUF)]
            + [pltpu.SemaphoreType.DMA for _ in range(2 * NBUF)]
        ),
    )
    dnt, mbt = run(vt, cidx)
    return jnp.swapaxes(dnt, 0, 1), jnp.swapaxes(mbt, 0, 1)
